# baseline (speedup 1.0000x reference)
import functools

import jax
import jax.numpy as jnp
from jax import lax
from jax.experimental import pallas as pl
from jax.experimental.pallas import tpu as pltpu
from jax.experimental.pallas import tpu_sc as plsc


def _table_prep(table_t):
    e, v = table_t.shape
    blk = 1024
    grid = ((v + blk - 1) // blk,)

    def k(a_ref, o_ref):
        eye = jnp.eye(e, dtype=jnp.float32)
        dn = (((0,), (0,)), ((), ()))
        lo = lax.dot_general(a_ref[:, :blk // 2], eye, dn,
                             preferred_element_type=jnp.float32)
        hi = lax.dot_general(a_ref[:, blk // 2:], eye, dn,
                             preferred_element_type=jnp.float32)
        o_ref[...] = jnp.concatenate([lo, hi], axis=1)

    return pl.pallas_call(
        k,
        grid=grid,
        in_specs=[pl.BlockSpec((e, blk), lambda i: (0, i))],
        out_specs=pl.BlockSpec((blk // 2, 2 * e), lambda i: (i, 0)),
        out_shape=jax.ShapeDtypeStruct((grid[0] * blk // 2, 2 * e), jnp.float32),
    )(table_t)


def _x_prep(x_t):
    b = x_t.shape[1]
    blk = 2048
    grid = (b // blk,)
    rowmap = (0, 2, 4, 1, 3, 5)

    def k(x_ref, o_ref):
        for j in range(6):
            v = x_ref[rowmap[j]].reshape(blk // 128, 128)
            o_ref[j] = (v & ~1023) + ((v & 511) << 1) + ((v >> 9) & 1)

    return pl.pallas_call(
        k,
        grid=grid,
        in_specs=[pl.BlockSpec((6, blk), lambda i: (0, i))],
        out_specs=pl.BlockSpec((6, blk // 128, 128), lambda i: (0, i, 0)),
        out_shape=jax.ShapeDtypeStruct((6, b // 128, 128), jnp.int32),
    )(x_t)


def _gather_rows(table, xp, batch):
    info = plsc.get_sparse_core_info()
    nw = info.num_cores * info.num_subcores
    n, e = 6 * batch, table.shape[1]
    per_w = n // nw
    chunk = 512
    n_ch = per_w // chunk

    mesh = plsc.VectorSubcoreMesh(core_axis_name="c", subcore_axis_name="s")

    half = chunk // 2

    @functools.partial(
        pl.kernel,
        mesh=mesh,
        compiler_params=pltpu.CompilerParams(use_tc_tiling_on_sc=False),
        out_type=jax.ShapeDtypeStruct((n // 2, 2, e), jnp.float32),
        scratch_types=[
            pltpu.VMEM((2, half), jnp.int32),
            pltpu.VMEM((2, half), jnp.int32),
            pltpu.VMEM((half, e), jnp.float32),
            pltpu.VMEM((half, e), jnp.float32),
            pltpu.VMEM((half, e), jnp.float32),
            pltpu.VMEM((half, e), jnp.float32),
            pltpu.SemaphoreType.DMA,
            pltpu.SemaphoreType.DMA,
        ],
    )
    def gather_k(table_hbm, xp_hbm, out_hbm, idx0, idx1,
                 ev0, od0, ev1, od1, sem0, sem1):
        wid = lax.axis_index("s") * info.num_cores + lax.axis_index("c")
        base = wid * per_w
        idxs = (idx0, idx1)
        bufs = ((ev0, od0), (ev1, od1))
        sems = (sem0, sem1)
        copies = [None, None]
        for c in range(n_ch + 1):
            if c < n_ch:
                s0 = base + c * chunk
                ct = s0 // (2 * batch)
                b0 = pl.multiple_of((s0 % (2 * batch)) // 2, half)
                idx_v = idxs[c % 2]
                pltpu.sync_copy(xp_hbm.at[pl.ds(ct * batch + b0, half)],
                                idx_v.at[0])
                pltpu.sync_copy(xp_hbm.at[pl.ds((3 + ct) * batch + b0, half)],
                                idx_v.at[1])
                ev_b, od_b = bufs[c % 2]
                cp = pltpu.async_copy(
                    table_hbm.at[idx_v.at[0]], ev_b, sems[c % 2])
                pltpu.async_copy(
                    table_hbm.at[idx_v.at[1]], od_b, sems[c % 2])
                copies[c % 2] = cp
            if c > 0:
                q0 = (base + (c - 1) * chunk) // 2
                ev_b, od_b = bufs[(c - 1) % 2]
                copies[(c - 1) % 2].wait()
                copies[(c - 1) % 2].wait()
                pltpu.sync_copy(ev_b, out_hbm.at[pl.ds(q0, half), 0])
                pltpu.sync_copy(od_b, out_hbm.at[pl.ds(q0, half), 1])

    return gather_k(table, xp)


def _mlp_panes(emb3, w1r, b1, w2, b2):
    _, b, _ = emb3.shape
    d_h = w1r.shape[2]
    d_out = w2.shape[1]
    blk = 2048

    def mlp_k(e_ref, w1_ref, b1_ref, w2_ref, b2_ref, o_ref):
        acc = b1_ref[...]
        for ct in range(3):
            acc = acc + jnp.dot(e_ref[ct], w1_ref[ct],
                                preferred_element_type=jnp.float32)
        h = jnp.tanh(acc)
        o_ref[...] = jnp.dot(h, w2_ref[...],
                             preferred_element_type=jnp.float32) + b2_ref[...]

    return pl.pallas_call(
        mlp_k,
        grid=(b // blk,),
        in_specs=[
            pl.BlockSpec((3, blk, 128), lambda i: (0, i, 0)),
            pl.BlockSpec((3, 128, d_h), lambda i: (0, 0, 0)),
            pl.BlockSpec((1, d_h), lambda i: (0, 0)),
            pl.BlockSpec((d_h, d_out), lambda i: (0, 0)),
            pl.BlockSpec((1, d_out), lambda i: (0, 0)),
        ],
        out_specs=pl.BlockSpec((blk, d_out), lambda i: (i, 0)),
        out_shape=jax.ShapeDtypeStruct((b, d_out), jnp.float32),
    )(emb3, w1r, b1.reshape(1, -1), w2, b2.reshape(1, -1))


def kernel(X, table, W1, b1, W2, b2):
    batch, n_words = X.shape
    vocab, embed = table.shape
    xp = _x_prep(X.T.astype(jnp.int32)).reshape(-1)
    t2 = _table_prep(table.T)
    rows = _gather_rows(t2.reshape(-1, embed), xp, batch)
    emb3 = rows.reshape(n_words // 2, batch, 2 * embed)
    w1r = W1.reshape(n_words // 2, 2 * embed, W1.shape[1])
    return _mlp_panes(emb3, w1r, b1, W2, b2)


# speedup vs baseline: 1.3187x; 1.3187x over previous
import functools

import jax
import jax.numpy as jnp
from jax import lax
from jax.experimental import pallas as pl
from jax.experimental.pallas import tpu as pltpu
from jax.experimental.pallas import tpu_sc as plsc


def _table_prep(table_t):
    e, v = table_t.shape
    blk = 2048
    grid = ((v + blk - 1) // blk,)

    def k(a_ref, o_ref):
        eye = jnp.eye(2 * e, dtype=jnp.float32)
        dn = (((0,), (0,)), ((), ()))
        stk = jnp.concatenate([a_ref[:, :blk // 2], a_ref[:, blk // 2:]],
                              axis=0)
        o_ref[...] = lax.dot_general(stk, eye, dn,
                                     preferred_element_type=jnp.float32)

    return pl.pallas_call(
        k,
        grid=grid,
        in_specs=[pl.BlockSpec((e, blk), lambda i: (0, i))],
        out_specs=pl.BlockSpec((blk // 2, 2 * e), lambda i: (i, 0)),
        out_shape=jax.ShapeDtypeStruct((grid[0] * blk // 2, 2 * e), jnp.float32),
    )(table_t)


def _x_prep(x_t):
    b = x_t.shape[1]
    blk = 2048
    grid = (b // blk,)
    rowmap = (0, 2, 4, 1, 3, 5)

    def k(x_ref, o_ref):
        for j in range(6):
            v = x_ref[rowmap[j]].reshape(blk // 128, 128)
            o_ref[j] = (v & ~2047) + ((v & 1023) << 1) + ((v >> 10) & 1)

    return pl.pallas_call(
        k,
        grid=grid,
        in_specs=[pl.BlockSpec((6, blk), lambda i: (0, i))],
        out_specs=pl.BlockSpec((6, blk // 128, 128), lambda i: (0, i, 0)),
        out_shape=jax.ShapeDtypeStruct((6, b // 128, 128), jnp.int32),
    )(x_t)


def _gather_rows(table, xp, batch):
    info = plsc.get_sparse_core_info()
    nw = info.num_cores * info.num_subcores
    n, e = 6 * batch, table.shape[1]
    per_w = n // nw
    chunk = 512
    n_ch = per_w // chunk

    mesh = plsc.VectorSubcoreMesh(core_axis_name="c", subcore_axis_name="s")

    half = chunk // 2

    @functools.partial(
        pl.kernel,
        mesh=mesh,
        compiler_params=pltpu.CompilerParams(use_tc_tiling_on_sc=False),
        out_type=jax.ShapeDtypeStruct((n // 2, 2, e), jnp.float32),
        scratch_types=[
            pltpu.VMEM((2, half), jnp.int32),
            pltpu.VMEM((2, half), jnp.int32),
            pltpu.VMEM((half, e), jnp.float32),
            pltpu.VMEM((half, e), jnp.float32),
            pltpu.VMEM((half, e), jnp.float32),
            pltpu.VMEM((half, e), jnp.float32),
            pltpu.SemaphoreType.DMA,
            pltpu.SemaphoreType.DMA,
        ],
    )
    def gather_k(table_hbm, xp_hbm, out_hbm, idx0, idx1,
                 ev0, od0, ev1, od1, sem0, sem1):
        wid = lax.axis_index("s") * info.num_cores + lax.axis_index("c")
        base = wid * per_w
        idxs = (idx0, idx1)
        bufs = ((ev0, od0), (ev1, od1))
        sems = (sem0, sem1)
        copies = [None, None]
        for c in range(n_ch + 1):
            if c < n_ch:
                s0 = base + c * chunk
                ct = s0 // (2 * batch)
                b0 = pl.multiple_of((s0 % (2 * batch)) // 2, half)
                idx_v = idxs[c % 2]
                pltpu.sync_copy(xp_hbm.at[pl.ds(ct * batch + b0, half)],
                                idx_v.at[0])
                pltpu.sync_copy(xp_hbm.at[pl.ds((3 + ct) * batch + b0, half)],
                                idx_v.at[1])
                ev_b, od_b = bufs[c % 2]
                cp = pltpu.async_copy(
                    table_hbm.at[idx_v.at[0]], ev_b, sems[c % 2])
                pltpu.async_copy(
                    table_hbm.at[idx_v.at[1]], od_b, sems[c % 2])
                copies[c % 2] = cp
            if c > 0:
                q0 = (base + (c - 1) * chunk) // 2
                ev_b, od_b = bufs[(c - 1) % 2]
                copies[(c - 1) % 2].wait()
                copies[(c - 1) % 2].wait()
                pltpu.sync_copy(ev_b, out_hbm.at[pl.ds(q0, half), 0])
                pltpu.sync_copy(od_b, out_hbm.at[pl.ds(q0, half), 1])

    return gather_k(table, xp)


def _mlp_panes(emb3, w1r, b1, w2, b2):
    _, b, _ = emb3.shape
    d_h = w1r.shape[2]
    d_out = w2.shape[1]
    blk = 2048

    def mlp_k(e_ref, w1_ref, b1_ref, w2_ref, b2_ref, o_ref):
        acc = b1_ref[...]
        for ct in range(3):
            acc = acc + jnp.dot(e_ref[ct], w1_ref[ct],
                                preferred_element_type=jnp.float32)
        h = jnp.tanh(acc)
        o_ref[...] = jnp.dot(h, w2_ref[...],
                             preferred_element_type=jnp.float32) + b2_ref[...]

    return pl.pallas_call(
        mlp_k,
        grid=(b // blk,),
        in_specs=[
            pl.BlockSpec((3, blk, 128), lambda i: (0, i, 0)),
            pl.BlockSpec((3, 128, d_h), lambda i: (0, 0, 0)),
            pl.BlockSpec((1, d_h), lambda i: (0, 0)),
            pl.BlockSpec((d_h, d_out), lambda i: (0, 0)),
            pl.BlockSpec((1, d_out), lambda i: (0, 0)),
        ],
        out_specs=pl.BlockSpec((blk, d_out), lambda i: (i, 0)),
        out_shape=jax.ShapeDtypeStruct((b, d_out), jnp.float32),
    )(emb3, w1r, b1.reshape(1, -1), w2, b2.reshape(1, -1))


def kernel(X, table, W1, b1, W2, b2):
    batch, n_words = X.shape
    vocab, embed = table.shape
    xp = _x_prep(X.T.astype(jnp.int32)).reshape(-1)
    t2 = _table_prep(table.T)
    rows = _gather_rows(t2.reshape(-1, embed), xp, batch)
    emb3 = rows.reshape(n_words // 2, batch, 2 * embed)
    w1r = W1.reshape(n_words // 2, 2 * embed, W1.shape[1])
    return _mlp_panes(emb3, w1r, b1, W2, b2)


# speedup vs baseline: 1.3455x; 1.0203x over previous
import functools

import jax
import jax.numpy as jnp
from jax import lax
from jax.experimental import pallas as pl
from jax.experimental.pallas import tpu as pltpu
from jax.experimental.pallas import tpu_sc as plsc


def _table_prep(table_t):
    e, v = table_t.shape
    blk = 2048
    grid = ((v + blk - 1) // blk,)

    def k(a_ref, o_ref):
        stk = jnp.concatenate([a_ref[:, :blk // 2], a_ref[:, blk // 2:]],
                              axis=0)
        o_ref[...] = jnp.transpose(stk)

    return pl.pallas_call(
        k,
        grid=grid,
        in_specs=[pl.BlockSpec((e, blk), lambda i: (0, i))],
        out_specs=pl.BlockSpec((blk // 2, 2 * e), lambda i: (i, 0)),
        out_shape=jax.ShapeDtypeStruct((grid[0] * blk // 2, 2 * e), jnp.float32),
    )(table_t)


def _x_prep(x_t):
    b = x_t.shape[1]
    blk = 2048
    grid = (b // blk,)
    rowmap = (0, 2, 4, 1, 3, 5)

    def k(x_ref, o_ref):
        for j in range(6):
            v = x_ref[rowmap[j]].reshape(blk // 128, 128)
            o_ref[j] = (v & ~2047) + ((v & 1023) << 1) + ((v >> 10) & 1)

    return pl.pallas_call(
        k,
        grid=grid,
        in_specs=[pl.BlockSpec((6, blk), lambda i: (0, i))],
        out_specs=pl.BlockSpec((6, blk // 128, 128), lambda i: (0, i, 0)),
        out_shape=jax.ShapeDtypeStruct((6, b // 128, 128), jnp.int32),
    )(x_t)


def _gather_rows(table, xp, batch):
    info = plsc.get_sparse_core_info()
    nw = info.num_cores * info.num_subcores
    n, e = 6 * batch, table.shape[1]
    per_w = n // nw
    chunk = 512
    n_ch = per_w // chunk

    mesh = plsc.VectorSubcoreMesh(core_axis_name="c", subcore_axis_name="s")

    half = chunk // 2

    @functools.partial(
        pl.kernel,
        mesh=mesh,
        compiler_params=pltpu.CompilerParams(use_tc_tiling_on_sc=False),
        out_type=jax.ShapeDtypeStruct((n // 2, 2, e), jnp.float32),
        scratch_types=[
            pltpu.VMEM((2, half), jnp.int32),
            pltpu.VMEM((2, half), jnp.int32),
            pltpu.VMEM((half, e), jnp.float32),
            pltpu.VMEM((half, e), jnp.float32),
            pltpu.VMEM((half, e), jnp.float32),
            pltpu.VMEM((half, e), jnp.float32),
            pltpu.SemaphoreType.DMA,
            pltpu.SemaphoreType.DMA,
        ],
    )
    def gather_k(table_hbm, xp_hbm, out_hbm, idx0, idx1,
                 ev0, od0, ev1, od1, sem0, sem1):
        wid = lax.axis_index("s") * info.num_cores + lax.axis_index("c")
        base = wid * per_w
        idxs = (idx0, idx1)
        bufs = ((ev0, od0), (ev1, od1))
        sems = (sem0, sem1)
        copies = [None, None]
        for c in range(n_ch + 1):
            if c < n_ch:
                s0 = base + c * chunk
                ct = s0 // (2 * batch)
                b0 = pl.multiple_of((s0 % (2 * batch)) // 2, half)
                idx_v = idxs[c % 2]
                pltpu.sync_copy(xp_hbm.at[pl.ds(ct * batch + b0, half)],
                                idx_v.at[0])
                pltpu.sync_copy(xp_hbm.at[pl.ds((3 + ct) * batch + b0, half)],
                                idx_v.at[1])
                ev_b, od_b = bufs[c % 2]
                cp = pltpu.async_copy(
                    table_hbm.at[idx_v.at[0]], ev_b, sems[c % 2])
                pltpu.async_copy(
                    table_hbm.at[idx_v.at[1]], od_b, sems[c % 2])
                copies[c % 2] = cp
            if c > 0:
                q0 = (base + (c - 1) * chunk) // 2
                ev_b, od_b = bufs[(c - 1) % 2]
                copies[(c - 1) % 2].wait()
                copies[(c - 1) % 2].wait()
                pltpu.sync_copy(ev_b, out_hbm.at[pl.ds(q0, half), 0])
                pltpu.sync_copy(od_b, out_hbm.at[pl.ds(q0, half), 1])

    return gather_k(table, xp)


def _mlp_panes(emb3, w1r, b1, w2, b2):
    _, b, _ = emb3.shape
    d_h = w1r.shape[2]
    d_out = w2.shape[1]
    blk = 2048

    def mlp_k(e_ref, w1_ref, b1_ref, w2_ref, b2_ref, o_ref):
        acc = b1_ref[...]
        for ct in range(3):
            acc = acc + jnp.dot(e_ref[ct], w1_ref[ct],
                                preferred_element_type=jnp.float32)
        h = jnp.tanh(acc)
        o_ref[...] = jnp.dot(h, w2_ref[...],
                             preferred_element_type=jnp.float32) + b2_ref[...]

    return pl.pallas_call(
        mlp_k,
        grid=(b // blk,),
        in_specs=[
            pl.BlockSpec((3, blk, 128), lambda i: (0, i, 0)),
            pl.BlockSpec((3, 128, d_h), lambda i: (0, 0, 0)),
            pl.BlockSpec((1, d_h), lambda i: (0, 0)),
            pl.BlockSpec((d_h, d_out), lambda i: (0, 0)),
            pl.BlockSpec((1, d_out), lambda i: (0, 0)),
        ],
        out_specs=pl.BlockSpec((blk, d_out), lambda i: (i, 0)),
        out_shape=jax.ShapeDtypeStruct((b, d_out), jnp.float32),
    )(emb3, w1r, b1.reshape(1, -1), w2, b2.reshape(1, -1))


def kernel(X, table, W1, b1, W2, b2):
    batch, n_words = X.shape
    vocab, embed = table.shape
    xp = _x_prep(X.T.astype(jnp.int32)).reshape(-1)
    t2 = _table_prep(table.T)
    rows = _gather_rows(t2.reshape(-1, embed), xp, batch)
    emb3 = rows.reshape(n_words // 2, batch, 2 * embed)
    w1r = W1.reshape(n_words // 2, 2 * embed, W1.shape[1])
    return _mlp_panes(emb3, w1r, b1, W2, b2)


# speedup vs baseline: 1.8385x; 1.3664x over previous
import functools

import jax
import jax.numpy as jnp
from jax import lax
from jax.experimental import pallas as pl
from jax.experimental.pallas import tpu as pltpu
from jax.experimental.pallas import tpu_sc as plsc


def _prep(table_t, x_t):
    e, v = table_t.shape
    b = x_t.shape[1]
    blk = 16384
    grid = ((v + blk - 1) // blk,)
    rowmap = (0, 2, 4, 1, 3, 5)

    def k(a_ref, x_ref, o_ref, xo_ref):
        stk = jnp.concatenate([a_ref[:, :blk // 2], a_ref[:, blk // 2:]],
                              axis=0)
        o_ref[...] = jnp.transpose(stk)

        @pl.when(pl.program_id(0) == 0)
        def _():
            for j in range(6):
                w = x_ref[rowmap[j]].reshape(b // 128, 128)
                xo_ref[j] = (w & ~16383) + ((w & 8191) << 1) + ((w >> 13) & 1)

    return pl.pallas_call(
        k,
        grid=grid,
        in_specs=[
            pl.BlockSpec((e, blk), lambda i: (0, i)),
            pl.BlockSpec((6, b), lambda i: (0, 0)),
        ],
        out_specs=[
            pl.BlockSpec((blk // 2, 2 * e), lambda i: (i, 0)),
            pl.BlockSpec((6, b // 128, 128), lambda i: (0, 0, 0)),
        ],
        out_shape=[
            jax.ShapeDtypeStruct((grid[0] * blk // 2, 2 * e), jnp.float32),
            jax.ShapeDtypeStruct((6, b // 128, 128), jnp.int32),
        ],
    )(table_t, x_t)


def _gather_rows(table, xp, batch):
    info = plsc.get_sparse_core_info()
    nw = info.num_cores * info.num_subcores
    n, e = 6 * batch, table.shape[1]
    per_w = n // nw
    chunk = 512
    n_ch = per_w // chunk

    mesh = plsc.VectorSubcoreMesh(core_axis_name="c", subcore_axis_name="s")

    half = chunk // 2

    @functools.partial(
        pl.kernel,
        mesh=mesh,
        compiler_params=pltpu.CompilerParams(use_tc_tiling_on_sc=False),
        out_type=jax.ShapeDtypeStruct((n // 2, 2, e), jnp.float32),
        scratch_types=[
            pltpu.VMEM((3, 2, half), jnp.int32),
            pltpu.VMEM((3, half, e), jnp.float32),
            pltpu.VMEM((3, half, e), jnp.float32),
            [pltpu.SemaphoreType.DMA] * 3,
            [pltpu.SemaphoreType.DMA] * 3,
            [pltpu.SemaphoreType.DMA] * 3,
        ],
    )
    def gather_k(table_hbm, xp_hbm, out_hbm, idx, ev, od, isem, gsem, ssem):
        wid = lax.axis_index("s") * info.num_cores + lax.axis_index("c")
        base = wid * per_w

        def idx_fetch(c):
            s0 = base + c * chunk
            ct = s0 // (2 * batch)
            b0 = pl.multiple_of((s0 % (2 * batch)) // 2, half)
            sl = c % 3
            cp0 = pltpu.async_copy(
                xp_hbm.at[pl.ds(ct * batch + b0, half)],
                idx.at[sl, 0], isem[sl])
            cp1 = pltpu.async_copy(
                xp_hbm.at[pl.ds((3 + ct) * batch + b0, half)],
                idx.at[sl, 1], isem[sl])
            return (cp0, cp1)

        idx_cps = {}
        g_cps = {}
        s_cps = {}
        idx_cps[0] = idx_fetch(0)
        if n_ch > 1:
            idx_cps[1] = idx_fetch(1)
        for c in range(n_ch + 1):
            if c >= 1:
                for cp in g_cps.pop(c - 1):
                    cp.wait()
                sl = (c - 1) % 3
                q0 = (base + (c - 1) * chunk) // 2
                s_cps[c - 1] = (
                    pltpu.async_copy(ev.at[sl], out_hbm.at[pl.ds(q0, half), 0],
                                     ssem[sl]),
                    pltpu.async_copy(od.at[sl], out_hbm.at[pl.ds(q0, half), 1],
                                     ssem[sl]),
                )
            if c < n_ch:
                if c >= 3:
                    for cp in s_cps.pop(c - 3):
                        cp.wait()
                sl = c % 3
                for cp in idx_cps.pop(c):
                    cp.wait()
                g_cps[c] = (
                    pltpu.async_copy(table_hbm.at[idx.at[sl, 0]], ev.at[sl],
                                     gsem[sl]),
                    pltpu.async_copy(table_hbm.at[idx.at[sl, 1]], od.at[sl],
                                     gsem[sl]),
                )
                if c + 2 < n_ch:
                    idx_cps[c + 2] = idx_fetch(c + 2)
        for k in sorted(s_cps):
            for cp in s_cps.pop(k):
                cp.wait()

    return gather_k(table, xp)


def _mlp_panes(emb3, w1r, b1, w2, b2):
    _, b, _ = emb3.shape
    d_h = w1r.shape[2]
    d_out = w2.shape[1]
    blk = 4096

    def mlp_k(e_ref, w1_ref, b1_ref, w2_ref, b2_ref, o_ref):
        acc = b1_ref[...]
        for ct in range(3):
            acc = acc + jnp.dot(e_ref[ct], w1_ref[ct],
                                preferred_element_type=jnp.float32)
        h = jnp.tanh(acc)
        o_ref[...] = jnp.dot(h, w2_ref[...],
                             preferred_element_type=jnp.float32) + b2_ref[...]

    return pl.pallas_call(
        mlp_k,
        grid=(b // blk,),
        in_specs=[
            pl.BlockSpec((3, blk, 128), lambda i: (0, i, 0)),
            pl.BlockSpec((3, 128, d_h), lambda i: (0, 0, 0)),
            pl.BlockSpec((1, d_h), lambda i: (0, 0)),
            pl.BlockSpec((d_h, d_out), lambda i: (0, 0)),
            pl.BlockSpec((1, d_out), lambda i: (0, 0)),
        ],
        out_specs=pl.BlockSpec((blk, d_out), lambda i: (i, 0)),
        out_shape=jax.ShapeDtypeStruct((b, d_out), jnp.float32),
    )(emb3, w1r, b1.reshape(1, -1), w2, b2.reshape(1, -1))


def kernel(X, table, W1, b1, W2, b2):
    batch, n_words = X.shape
    vocab, embed = table.shape
    t2, xp3 = _prep(table.T, X.T.astype(jnp.int32))
    xp = xp3.reshape(-1)
    rows = _gather_rows(t2.reshape(-1, embed), xp, batch)
    emb3 = rows.reshape(n_words // 2, batch, 2 * embed)
    w1r = W1.reshape(n_words // 2, 2 * embed, W1.shape[1])
    return _mlp_panes(emb3, w1r, b1, W2, b2)
